# baseline (speedup 1.0000x reference)
import functools
import jax
import jax.numpy as jnp
from jax import lax
from jax.experimental import pallas as pl
from jax.experimental.pallas import tpu as pltpu
from jax.experimental.pallas import tpu_sc as plsc

_NC = 2
_NS = 16
_NW = _NC * _NS
_L = 16
_CHUNK = 32


def kernel(token_embeddings, pos_table):
    batch, seq, dim = token_embeddings.shape
    tok_flat = token_embeddings.reshape(batch * seq * dim)
    pos_flat = pos_table.reshape(seq * dim)

    seq_per_w = seq // _NW
    steps = seq_per_w // _CHUNK
    chunk_words = _CHUNK * dim

    mesh = plsc.VectorSubcoreMesh(
        core_axis_name="c", subcore_axis_name="s",
        num_cores=_NC, num_subcores=_NS,
    )

    @functools.partial(
        pl.kernel,
        out_type=jax.ShapeDtypeStruct((batch * seq * dim,), jnp.float32),
        mesh=mesh,
        scratch_types=[
            pltpu.VMEM((chunk_words,), jnp.float32),
            pltpu.VMEM((chunk_words,), jnp.float32),
        ],
    )
    def sc_add(tok_hbm, pos_hbm, out_hbm, pbuf, obuf):
        w = lax.axis_index("s") * _NC + lax.axis_index("c")
        seq0 = w * seq_per_w

        def step_body(t, carry):
            s0 = seq0 + t * _CHUNK
            pltpu.sync_copy(pos_hbm.at[pl.ds(s0 * dim, chunk_words)], pbuf)

            def batch_body(b, carry_b):
                off = (b * seq + s0) * dim
                pltpu.sync_copy(tok_hbm.at[pl.ds(off, chunk_words)], obuf)

                @plsc.parallel_loop(0, chunk_words, step=_L, unroll=8)
                def vbody(i):
                    plsc.addupdate(obuf.at[pl.ds(i, _L)], pbuf[pl.ds(i, _L)])

                pltpu.sync_copy(obuf, out_hbm.at[pl.ds(off, chunk_words)])
                return carry_b

            lax.fori_loop(0, batch, batch_body, 0)
            return carry

        lax.fori_loop(0, steps, step_body, 0)

    out = sc_add(tok_flat, pos_flat)
    return out.reshape(batch, seq, dim)


# speedup vs baseline: 1.0471x; 1.0471x over previous
import functools
import jax
import jax.numpy as jnp
from jax import lax
from jax.experimental import pallas as pl
from jax.experimental.pallas import tpu as pltpu
from jax.experimental.pallas import tpu_sc as plsc

_NC = 2
_NS = 16
_NW = _NC * _NS
_L = 16
_CHUNK = 16
_NBUF = 4


def kernel(token_embeddings, pos_table):
    batch, seq, dim = token_embeddings.shape
    tok_flat = token_embeddings.reshape(batch * seq * dim)
    pos_flat = pos_table.reshape(seq * dim)

    seq_per_w = seq // _NW
    steps = seq_per_w // _CHUNK
    chunk_words = _CHUNK * dim
    n_items = steps * batch

    mesh = plsc.VectorSubcoreMesh(
        core_axis_name="c", subcore_axis_name="s",
        num_cores=_NC, num_subcores=_NS,
    )

    @functools.partial(
        pl.kernel,
        out_type=jax.ShapeDtypeStruct((batch * seq * dim,), jnp.float32),
        mesh=mesh,
        scratch_types=[
            pltpu.VMEM((2, chunk_words), jnp.float32),
            pltpu.VMEM((_NBUF, chunk_words), jnp.float32),
            pltpu.SemaphoreType.DMA,
            pltpu.SemaphoreType.DMA,
            pltpu.SemaphoreType.DMA,
            pltpu.SemaphoreType.DMA,
            pltpu.SemaphoreType.DMA,
            pltpu.SemaphoreType.DMA,
            pltpu.SemaphoreType.DMA,
            pltpu.SemaphoreType.DMA,
            pltpu.SemaphoreType.DMA,
            pltpu.SemaphoreType.DMA,
        ],
    )
    def sc_add(tok_hbm, pos_hbm, out_hbm, pbuf, obuf, psem0, psem1,
               isem0, isem1, isem2, isem3, osem0, osem1, osem2, osem3):
        w = lax.axis_index("s") * _NC + lax.axis_index("c")
        seq0 = w * seq_per_w
        psems = (psem0, psem1)
        isems = (isem0, isem1, isem2, isem3)
        osems = (osem0, osem1, osem2, osem3)

        def hbm_off(k):
            t, b = divmod(k, batch)
            return (b * seq + seq0 + t * _CHUNK) * dim

        def start_in(k):
            return pltpu.async_copy(
                tok_hbm.at[pl.ds(hbm_off(k), chunk_words)],
                obuf.at[k % _NBUF], isems[k % _NBUF])

        def start_pos(t):
            return pltpu.async_copy(
                pos_hbm.at[pl.ds((seq0 + t * _CHUNK) * dim, chunk_words)],
                pbuf.at[t % 2], psems[t % 2])

        pos_dma = {0: start_pos(0), 1: start_pos(1)}
        in_dma = {0: start_in(0), 1: start_in(1)}
        out_dma = {}

        for k in range(n_items):
            t, b = divmod(k, batch)
            buf = k % _NBUF
            if b == 0:
                pos_dma.pop(t).wait()
            in_dma.pop(k).wait()

            @plsc.parallel_loop(0, chunk_words, step=_L, unroll=8)
            def vbody(i):
                plsc.addupdate(obuf.at[buf, pl.ds(i, _L)],
                               pbuf[t % 2, pl.ds(i, _L)])

            out_dma[k] = pltpu.async_copy(
                obuf.at[buf], out_hbm.at[pl.ds(hbm_off(k), chunk_words)],
                osems[buf])
            if b == batch - 1 and t + 2 < steps:
                pos_dma[t + 2] = start_pos(t + 2)
            if k + 2 < n_items:
                if k - 2 >= 0:
                    out_dma.pop(k - 2).wait()
                in_dma[k + 2] = start_in(k + 2)

        for d in out_dma.values():
            d.wait()

    out = sc_add(tok_flat, pos_flat)
    return out.reshape(batch, seq, dim)


# speedup vs baseline: 1.2287x; 1.1734x over previous
import functools
import jax
import jax.numpy as jnp
from jax import lax
from jax.experimental import pallas as pl
from jax.experimental.pallas import tpu as pltpu
from jax.experimental.pallas import tpu_sc as plsc

_NC = 2
_NS = 16
_NW = _NC * _NS
_L = 16
_CHUNK = 16
_NBUF = 4
_BS = 512
_SPLIT = 3


def _tc_add_body(tok_ref, pos_ref, out_ref):
    out_ref[...] = tok_ref[...] + pos_ref[...]


def _tc_part(token_embeddings, pos_table):
    batch, seq, dim = token_embeddings.shape
    grid = (seq // _BS, batch)
    return pl.pallas_call(
        _tc_add_body,
        grid=grid,
        in_specs=[
            pl.BlockSpec((1, _BS, dim), lambda s, b: (b, s, 0)),
            pl.BlockSpec((_BS, dim), lambda s, b: (s, 0)),
        ],
        out_specs=pl.BlockSpec((1, _BS, dim), lambda s, b: (b, s, 0)),
        out_shape=jax.ShapeDtypeStruct((batch, seq, dim), token_embeddings.dtype),
    )(token_embeddings, pos_table)


def _sc_part(token_embeddings, pos_table):
    batch, seq, dim = token_embeddings.shape
    tok_flat = token_embeddings.reshape(batch * seq * dim)
    pos_flat = pos_table.reshape(seq * dim)

    seq_per_w = seq // _NW
    steps = seq_per_w // _CHUNK
    chunk_words = _CHUNK * dim
    n_items = steps * batch

    mesh = plsc.VectorSubcoreMesh(
        core_axis_name="c", subcore_axis_name="s",
        num_cores=_NC, num_subcores=_NS,
    )

    @functools.partial(
        pl.kernel,
        out_type=jax.ShapeDtypeStruct((batch * seq * dim,), jnp.float32),
        mesh=mesh,
        scratch_types=[
            pltpu.VMEM((2, chunk_words), jnp.float32),
            pltpu.VMEM((_NBUF, chunk_words), jnp.float32),
            pltpu.SemaphoreType.DMA,
            pltpu.SemaphoreType.DMA,
            pltpu.SemaphoreType.DMA,
            pltpu.SemaphoreType.DMA,
            pltpu.SemaphoreType.DMA,
            pltpu.SemaphoreType.DMA,
            pltpu.SemaphoreType.DMA,
            pltpu.SemaphoreType.DMA,
            pltpu.SemaphoreType.DMA,
            pltpu.SemaphoreType.DMA,
        ],
    )
    def sc_add(tok_hbm, pos_hbm, out_hbm, pbuf, obuf, psem0, psem1,
               isem0, isem1, isem2, isem3, osem0, osem1, osem2, osem3):
        w = lax.axis_index("s") * _NC + lax.axis_index("c")
        seq0 = w * seq_per_w
        psems = (psem0, psem1)
        isems = (isem0, isem1, isem2, isem3)
        osems = (osem0, osem1, osem2, osem3)

        def hbm_off(k):
            t, b = divmod(k, batch)
            return (b * seq + seq0 + t * _CHUNK) * dim

        def start_in(k):
            return pltpu.async_copy(
                tok_hbm.at[pl.ds(hbm_off(k), chunk_words)],
                obuf.at[k % _NBUF], isems[k % _NBUF])

        def start_pos(t):
            return pltpu.async_copy(
                pos_hbm.at[pl.ds((seq0 + t * _CHUNK) * dim, chunk_words)],
                pbuf.at[t % 2], psems[t % 2])

        pos_dma = {0: start_pos(0), 1: start_pos(1)}
        in_dma = {0: start_in(0), 1: start_in(1)}
        out_dma = {}

        for k in range(n_items):
            t, b = divmod(k, batch)
            buf = k % _NBUF
            if b == 0:
                pos_dma.pop(t).wait()
            in_dma.pop(k).wait()

            @plsc.parallel_loop(0, chunk_words, step=_L, unroll=8)
            def vbody(i):
                plsc.addupdate(obuf.at[buf, pl.ds(i, _L)],
                               pbuf[t % 2, pl.ds(i, _L)])

            out_dma[k] = pltpu.async_copy(
                obuf.at[buf], out_hbm.at[pl.ds(hbm_off(k), chunk_words)],
                osems[buf])
            if b == batch - 1 and t + 2 < steps:
                pos_dma[t + 2] = start_pos(t + 2)
            if k + 2 < n_items:
                if k - 2 >= 0:
                    out_dma.pop(k - 2).wait()
                in_dma[k + 2] = start_in(k + 2)

        for d in out_dma.values():
            d.wait()

    out = sc_add(tok_flat, pos_flat)
    return out.reshape(batch, seq, dim)


def kernel(token_embeddings, pos_table):
    tc_out = _tc_part(token_embeddings[:_SPLIT], pos_table)
    sc_out = _sc_part(token_embeddings[_SPLIT:], pos_table)
    return jnp.concatenate([tc_out, sc_out], axis=0)


# speedup vs baseline: 1.2901x; 1.0499x over previous
import functools
import jax
import jax.numpy as jnp
from jax import lax
from jax.experimental import pallas as pl
from jax.experimental.pallas import tpu as pltpu
from jax.experimental.pallas import tpu_sc as plsc

_NC = 2
_NS = 16
_NW = _NC * _NS
_L = 16
_CHUNK = 16
_NBUF = 4
_BS = 512
_SPLIT = 3


def _tc_add_body(tok_ref, pos_ref, out_ref):
    out_ref[...] = tok_ref[...] + pos_ref[...]


def _tc_part(token_embeddings, pos_table, nbatch):
    _, seq, dim = token_embeddings.shape
    grid = (seq // _BS, nbatch)
    return pl.pallas_call(
        _tc_add_body,
        grid=grid,
        in_specs=[
            pl.BlockSpec((1, _BS, dim), lambda s, b: (b, s, 0)),
            pl.BlockSpec((_BS, dim), lambda s, b: (s, 0)),
        ],
        out_specs=pl.BlockSpec((1, _BS, dim), lambda s, b: (b, s, 0)),
        out_shape=jax.ShapeDtypeStruct((nbatch, seq, dim), token_embeddings.dtype),
    )(token_embeddings, pos_table)


def _sc_part(token_embeddings, pos_table, batch0):
    full_batch, seq, dim = token_embeddings.shape
    batch = full_batch - batch0
    tok_flat = token_embeddings.reshape(full_batch * seq * dim)
    pos_flat = pos_table.reshape(seq * dim)

    seq_per_w = seq // _NW
    steps = seq_per_w // _CHUNK
    chunk_words = _CHUNK * dim
    n_items = steps * batch

    mesh = plsc.VectorSubcoreMesh(
        core_axis_name="c", subcore_axis_name="s",
        num_cores=_NC, num_subcores=_NS,
    )

    @functools.partial(
        pl.kernel,
        out_type=jax.ShapeDtypeStruct((batch * seq * dim,), jnp.float32),
        mesh=mesh,
        scratch_types=[
            pltpu.VMEM((2, chunk_words), jnp.float32),
            pltpu.VMEM((_NBUF, chunk_words), jnp.float32),
            pltpu.SemaphoreType.DMA,
            pltpu.SemaphoreType.DMA,
            pltpu.SemaphoreType.DMA,
            pltpu.SemaphoreType.DMA,
            pltpu.SemaphoreType.DMA,
            pltpu.SemaphoreType.DMA,
            pltpu.SemaphoreType.DMA,
            pltpu.SemaphoreType.DMA,
            pltpu.SemaphoreType.DMA,
            pltpu.SemaphoreType.DMA,
        ],
    )
    def sc_add(tok_hbm, pos_hbm, out_hbm, pbuf, obuf, psem0, psem1,
               isem0, isem1, isem2, isem3, osem0, osem1, osem2, osem3):
        w = lax.axis_index("s") * _NC + lax.axis_index("c")
        seq0 = w * seq_per_w
        psems = (psem0, psem1)
        isems = (isem0, isem1, isem2, isem3)
        osems = (osem0, osem1, osem2, osem3)

        def hbm_off(k):
            t, b = divmod(k, batch)
            return (b * seq + seq0 + t * _CHUNK) * dim

        def start_in(k):
            return pltpu.async_copy(
                tok_hbm.at[pl.ds(hbm_off(k) + batch0 * seq * dim, chunk_words)],
                obuf.at[k % _NBUF], isems[k % _NBUF])

        def start_pos(t):
            return pltpu.async_copy(
                pos_hbm.at[pl.ds((seq0 + t * _CHUNK) * dim, chunk_words)],
                pbuf.at[t % 2], psems[t % 2])

        pos_dma = {0: start_pos(0), 1: start_pos(1)}
        in_dma = {0: start_in(0), 1: start_in(1)}
        out_dma = {}

        for k in range(n_items):
            t, b = divmod(k, batch)
            buf = k % _NBUF
            if b == 0:
                pos_dma.pop(t).wait()
            in_dma.pop(k).wait()

            @plsc.parallel_loop(0, chunk_words, step=_L, unroll=8)
            def vbody(i):
                plsc.addupdate(obuf.at[buf, pl.ds(i, _L)],
                               pbuf[t % 2, pl.ds(i, _L)])

            out_dma[k] = pltpu.async_copy(
                obuf.at[buf], out_hbm.at[pl.ds(hbm_off(k), chunk_words)],
                osems[buf])
            if b == batch - 1 and t + 2 < steps:
                pos_dma[t + 2] = start_pos(t + 2)
            if k + 2 < n_items:
                if k - 2 >= 0:
                    out_dma.pop(k - 2).wait()
                in_dma[k + 2] = start_in(k + 2)

        for d in out_dma.values():
            d.wait()

    out = sc_add(tok_flat, pos_flat)
    return out.reshape(batch, seq, dim)


def kernel(token_embeddings, pos_table):
    tc_out = _tc_part(token_embeddings, pos_table, _SPLIT)
    sc_out = _sc_part(token_embeddings, pos_table, _SPLIT)
    return jnp.concatenate([tc_out, sc_out], axis=0)


# speedup vs baseline: 5.1116x; 3.9623x over previous
import jax
import jax.numpy as jnp
from jax.experimental import pallas as pl

_BS = 512


def _add_body(tok_ref, pos_ref, out_ref):
    out_ref[...] = tok_ref[...] + pos_ref[...][None, :, :]


def kernel(token_embeddings, pos_table):
    batch, seq, dim = token_embeddings.shape
    return pl.pallas_call(
        _add_body,
        grid=(seq // _BS,),
        in_specs=[
            pl.BlockSpec((batch, _BS, dim), lambda s: (0, s, 0)),
            pl.BlockSpec((_BS, dim), lambda s: (s, 0)),
        ],
        out_specs=pl.BlockSpec((batch, _BS, dim), lambda s: (0, s, 0)),
        out_shape=jax.ShapeDtypeStruct((batch, seq, dim), token_embeddings.dtype),
    )(token_embeddings, pos_table)
